# Initial kernel scaffold; baseline (speedup 1.0000x reference)
import jax
import jax.numpy as jnp
from jax import lax
from jax.experimental import pallas as pl
from jax.experimental.pallas import tpu as pltpu
from jax.experimental.pallas import tpu_sc as plsc

N = 10000
E = 320000
D = 128
H = 128
T = 9

NC = 2
NS = 16
NW = NC * NS

CH = 128
EROWS = 2560
E_PAD = EROWS * CH
ROWS_PER_W = EROWS // NW
ACC_N = 10240
PAD_ROW = ACC_N - 1
ZROWS = 64
DEG_W = 16


def _mesh():
    return plsc.VectorSubcoreMesh(
        core_axis_name="c", subcore_axis_name="s", num_cores=NC, num_subcores=NS
    )


def _deg_body(dst_hbm, ones_hbm, zeros_hbm, deg_hbm, acc, dst_v, ones_v, zv):
    c = lax.axis_index("c")
    s = lax.axis_index("s")
    wid = s * NC + c
    pltpu.sync_copy(dst_hbm.at[pl.ds(wid * ROWS_PER_W, ROWS_PER_W)], dst_v)
    pltpu.sync_copy(ones_hbm, ones_v)
    pltpu.sync_copy(zeros_hbm, zv)
    rows_per_s = ACC_N // NS
    pltpu.sync_copy(zv, acc.at[pl.ds(s * rows_per_s, rows_per_s)])
    plsc.subcore_barrier()

    def chunk(j, carry):
        pltpu.sync_copy(ones_v, acc.at[dst_v.at[j]], add=True)
        return carry

    lax.fori_loop(0, ROWS_PER_W, chunk, 0)
    plsc.subcore_barrier()
    pltpu.sync_copy(
        acc.at[pl.ds(s * rows_per_s, rows_per_s)],
        deg_hbm.at[c, pl.ds(s * rows_per_s, rows_per_s)],
    )


def _make_deg_kernel():
    return pl.kernel(
        _deg_body,
        out_type=jax.ShapeDtypeStruct((NC, ACC_N, DEG_W), jnp.float32),
        mesh=_mesh(),
        scratch_types=[
            pltpu.VMEM_SHARED((ACC_N, DEG_W), jnp.float32),
            pltpu.VMEM((ROWS_PER_W, CH), jnp.int32),
            pltpu.VMEM((CH, DEG_W), jnp.float32),
            pltpu.VMEM((ACC_N // NS, DEG_W), jnp.float32),
        ],
    )


def _scatter_body(src_hbm, dst_hbm, *rest):
    y_hbm = rest[:T]
    zrow_hbm = rest[T]
    s_out = rest[T + 1]
    acc, src_v, dst_v, buf0, buf1, zbuf, sem0, sem1 = rest[T + 2:]

    c = lax.axis_index("c")
    s = lax.axis_index("s")
    wid = s * NC + c
    base = wid * ROWS_PER_W
    pltpu.sync_copy(src_hbm.at[pl.ds(base, ROWS_PER_W)], src_v)
    pltpu.sync_copy(dst_hbm.at[pl.ds(base, ROWS_PER_W)], dst_v)
    pltpu.sync_copy(zrow_hbm, zbuf)

    zrows_per_s = ACC_N // NS
    out_rows_per_s = N // NS
    half = ROWS_PER_W // 2

    for t in range(T):
        yt = y_hbm[t]

        def zero(k, carry):
            pltpu.sync_copy(zbuf, acc.at[pl.ds(s * zrows_per_s + k * ZROWS, ZROWS)])
            return carry

        lax.fori_loop(0, zrows_per_s // ZROWS, zero, 0)
        plsc.subcore_barrier()

        pltpu.async_copy(yt.at[src_v.at[0]], buf0, sem0)

        def chunk(k, carry):
            j0 = 2 * k
            pltpu.make_async_copy(yt, buf0, sem0).wait()
            pltpu.async_copy(yt.at[src_v.at[j0 + 1]], buf1, sem1)
            pltpu.sync_copy(buf0, acc.at[dst_v.at[j0]], add=True)
            pltpu.make_async_copy(yt, buf1, sem1).wait()

            @pl.when(k < half - 1)
            def _():
                pltpu.async_copy(yt.at[src_v.at[j0 + 2]], buf0, sem0)

            pltpu.sync_copy(buf1, acc.at[dst_v.at[j0 + 1]], add=True)
            return carry

        lax.fori_loop(0, half, chunk, 0)
        plsc.subcore_barrier()
        pltpu.sync_copy(
            acc.at[pl.ds(s * out_rows_per_s, out_rows_per_s)],
            s_out.at[t, c, pl.ds(s * out_rows_per_s, out_rows_per_s)],
        )
        plsc.subcore_barrier()


def _make_scatter_kernel():
    return pl.kernel(
        _scatter_body,
        out_type=jax.ShapeDtypeStruct((T, NC, N, D), jnp.float32),
        mesh=_mesh(),
        scratch_types=[
            pltpu.VMEM_SHARED((ACC_N, D), jnp.float32),
            pltpu.VMEM((ROWS_PER_W, CH), jnp.int32),
            pltpu.VMEM((ROWS_PER_W, CH), jnp.int32),
            pltpu.VMEM((CH, D), jnp.float32),
            pltpu.VMEM((CH, D), jnp.float32),
            pltpu.VMEM((ZROWS, D), jnp.float32),
            pltpu.SemaphoreType.DMA,
            pltpu.SemaphoreType.DMA,
        ],
    )


RB = 500


def _dinv_from_deg(deg_ref):
    deg = deg_ref[0, :, 0] + deg_ref[1, :, 0] + 1.0
    return lax.rsqrt(jnp.maximum(deg, 1.0))


def _xw_body(x_ref, w_ref, deg_ref, *y_refs):
    dinv = _dinv_from_deg(deg_ref)[:, None]
    w = w_ref[...]
    for t in range(T):
        y_refs[t][...] = dinv * jnp.dot(
            x_ref[t], w, preferred_element_type=jnp.float32
        )


def _make_xw_call():
    return pl.pallas_call(
        _xw_body,
        grid=(N // RB,),
        in_specs=[
            pl.BlockSpec((T, RB, D), lambda i: (0, i, 0)),
            pl.BlockSpec((D, H), lambda i: (0, 0)),
            pl.BlockSpec((NC, RB, DEG_W), lambda i: (0, i, 0)),
        ],
        out_specs=[pl.BlockSpec((RB, H), lambda i: (i, 0)) for _ in range(T)],
        out_shape=[jax.ShapeDtypeStruct((N, H), jnp.float32) for _ in range(T)],
    )


RD = 500


def _lstm_body(s_ref, deg_ref, bg_ref, wih_ref, whh_ref, bih_ref, bhh_ref,
               wfc_ref, bfc_ref, *rest):
    y_refs = rest[:T]
    out_ref = rest[T]
    dinv = _dinv_from_deg(deg_ref)[:, None]
    bg = bg_ref[...]
    wih = wih_ref[...]
    whh = whh_ref[...]
    bias = bih_ref[...] + bhh_ref[...]
    wfc = wfc_ref[...]
    bfc = bfc_ref[0, 0]

    h = jnp.zeros((RD, H), jnp.float32)
    c = jnp.zeros((RD, H), jnp.float32)
    cols = []
    for t in range(T):
        ssum = s_ref[t, 0] + s_ref[t, 1] + y_refs[t][...]
        emb = jnp.maximum(dinv * ssum + bg, 0.0)
        gates = (
            jnp.dot(emb, wih, preferred_element_type=jnp.float32)
            + jnp.dot(h, whh, preferred_element_type=jnp.float32)
            + bias
        )
        gi = jax.nn.sigmoid(gates[:, 0:H])
        gf = jax.nn.sigmoid(gates[:, H:2 * H])
        gg = jnp.tanh(gates[:, 2 * H:3 * H])
        go = jax.nn.sigmoid(gates[:, 3 * H:4 * H])
        c = gf * c + gi * gg
        h = go * jnp.tanh(c)
        cols.append(jnp.dot(h, wfc, preferred_element_type=jnp.float32) + bfc)
    out_ref[...] = jnp.concatenate(cols, axis=1)


def _make_lstm_call():
    return pl.pallas_call(
        _lstm_body,
        grid=(N // RD,),
        in_specs=(
            [
                pl.BlockSpec((T, NC, RD, H), lambda i: (0, 0, i, 0)),
                pl.BlockSpec((NC, RD, DEG_W), lambda i: (0, i, 0)),
                pl.BlockSpec((1, H), lambda i: (0, 0)),
                pl.BlockSpec((H, 4 * H), lambda i: (0, 0)),
                pl.BlockSpec((H, 4 * H), lambda i: (0, 0)),
                pl.BlockSpec((1, 4 * H), lambda i: (0, 0)),
                pl.BlockSpec((1, 4 * H), lambda i: (0, 0)),
                pl.BlockSpec((H, 1), lambda i: (0, 0)),
                pl.BlockSpec((1, 1), lambda i: (0, 0)),
            ]
            + [pl.BlockSpec((RD, H), lambda i: (i, 0)) for _ in range(T)]
        ),
        out_specs=pl.BlockSpec((RD, T), lambda i: (i, 0)),
        out_shape=jax.ShapeDtypeStruct((N, T), jnp.float32),
    )


@jax.jit
def kernel(x_seq, edge_index, W_gcn, b_gcn, W_ih, W_hh, b_ih, b_hh, W_fc, b_fc):
    src = edge_index[0]
    dst = edge_index[1]
    npad = E_PAD - E
    src_p = jnp.concatenate([src, jnp.zeros((npad,), jnp.int32)]).reshape(EROWS, CH)
    dst_p = jnp.concatenate(
        [dst, jnp.full((npad,), PAD_ROW, jnp.int32)]
    ).reshape(EROWS, CH)

    ones16 = jnp.ones((CH, DEG_W), jnp.float32)
    zdeg = jnp.zeros((ACC_N // NS, DEG_W), jnp.float32)
    zrow = jnp.zeros((ZROWS, D), jnp.float32)

    deg = _make_deg_kernel()(dst_p, ones16, zdeg)
    ys = _make_xw_call()(x_seq, W_gcn, deg)
    s_parts = _make_scatter_kernel()(src_p, dst_p, *ys, zrow)
    preds = _make_lstm_call()(
        s_parts,
        deg,
        b_gcn.reshape(1, H),
        W_ih.T,
        W_hh.T,
        b_ih.reshape(1, 4 * H),
        b_hh.reshape(1, 4 * H),
        W_fc.T,
        b_fc.reshape(1, 1),
        *ys,
    )
    return preds.reshape(N, T, 1)


# speedup vs baseline: 5.0806x; 5.0806x over previous
import jax
import jax.numpy as jnp
from jax import lax
from jax.experimental import pallas as pl
from jax.experimental.pallas import tpu as pltpu
from jax.experimental.pallas import tpu_sc as plsc

N = 10000
E = 320000
D = 128
H = 128
T = 9

NC = 2
NS = 16
NW = NC * NS

CH = 128
EROWS = 2560
E_PAD = EROWS * CH
ROWS_PER_W = EROWS // NW
ACC_N = 10240
PAD_ROW = ACC_N - 1
ZROWS = 16
DEG_W = 128


def _mesh():
    return plsc.VectorSubcoreMesh(
        core_axis_name="c", subcore_axis_name="s", num_cores=NC, num_subcores=NS
    )


def _deg_body(dst_hbm, ones_hbm, zeros_hbm, deg_hbm, acc, dst_v, ones_v, zv):
    c = lax.axis_index("c")
    s = lax.axis_index("s")
    wid = s * NC + c
    pltpu.sync_copy(dst_hbm.at[pl.ds(wid * ROWS_PER_W, ROWS_PER_W)], dst_v)
    pltpu.sync_copy(ones_hbm, ones_v)
    pltpu.sync_copy(zeros_hbm, zv)
    rows_per_s = ACC_N // NS

    def zero(k, carry):
        pltpu.sync_copy(zv, acc.at[pl.ds(s * rows_per_s + k * ZROWS, ZROWS)])
        return carry

    lax.fori_loop(0, rows_per_s // ZROWS, zero, 0)
    plsc.subcore_barrier()

    def chunk(j, carry):
        pltpu.sync_copy(ones_v, acc.at[dst_v.at[j]], add=True)
        return carry

    lax.fori_loop(0, ROWS_PER_W, chunk, 0)
    plsc.subcore_barrier()
    pltpu.sync_copy(
        acc.at[pl.ds(s * rows_per_s, rows_per_s)],
        deg_hbm.at[c, pl.ds(s * rows_per_s, rows_per_s)],
    )


def _make_deg_kernel():
    return pl.kernel(
        _deg_body,
        out_type=jax.ShapeDtypeStruct((NC, ACC_N, DEG_W), jnp.float32),
        mesh=_mesh(),
        scratch_types=[
            pltpu.VMEM_SHARED((ACC_N, DEG_W), jnp.float32),
            pltpu.VMEM((ROWS_PER_W, CH), jnp.int32),
            pltpu.VMEM((CH, DEG_W), jnp.float32),
            pltpu.VMEM((ZROWS, DEG_W), jnp.float32),
        ],
    )


def _scatter_body(src_hbm, dst_hbm, *rest):
    y_hbm = rest[:T]
    zrow_hbm = rest[T]
    s_out = rest[T + 1]
    acc, src_v, dst_v, buf0, zbuf, sem0 = rest[T + 2:]

    c = lax.axis_index("c")
    s = lax.axis_index("s")
    wid = s * NC + c
    base = wid * ROWS_PER_W
    pltpu.sync_copy(src_hbm.at[pl.ds(base, ROWS_PER_W)], src_v)
    pltpu.sync_copy(dst_hbm.at[pl.ds(base, ROWS_PER_W)], dst_v)
    pltpu.sync_copy(zrow_hbm, zbuf)

    zrows_per_s = ACC_N // NS

    for t in range(T):
        yt = y_hbm[t]

        def zero(k, carry):
            pltpu.sync_copy(zbuf, acc.at[pl.ds(s * zrows_per_s + k * ZROWS, ZROWS)])
            return carry

        lax.fori_loop(0, zrows_per_s // ZROWS, zero, 0)
        plsc.subcore_barrier()

        def chunk(j, carry):
            pltpu.async_copy(yt.at[src_v.at[j]], buf0, sem0).wait()
            pltpu.sync_copy(buf0, acc.at[dst_v.at[j]], add=True)
            return carry

        lax.fori_loop(0, ROWS_PER_W, chunk, 0)
        plsc.subcore_barrier()
        pltpu.sync_copy(
            acc.at[pl.ds(s * zrows_per_s, zrows_per_s)],
            s_out.at[t, c, pl.ds(s * zrows_per_s, zrows_per_s)],
        )
        plsc.subcore_barrier()


def _make_scatter_kernel():
    return pl.kernel(
        _scatter_body,
        out_type=jax.ShapeDtypeStruct((T, NC, ACC_N, D), jnp.float32),
        mesh=_mesh(),
        scratch_types=[
            pltpu.VMEM_SHARED((ACC_N, D), jnp.float32),
            pltpu.VMEM((ROWS_PER_W, CH), jnp.int32),
            pltpu.VMEM((ROWS_PER_W, CH), jnp.int32),
            pltpu.VMEM((CH, D), jnp.float32),
            pltpu.VMEM((ZROWS, D), jnp.float32),
            pltpu.SemaphoreType.DMA,
        ],
    )


RB = 400


def _dinv_from_deg(deg_ref):
    deg = deg_ref[0, :, 0] + deg_ref[1, :, 0] + 1.0
    return lax.rsqrt(jnp.maximum(deg, 1.0))


def _xw_body(x_ref, w_ref, deg_ref, *y_refs):
    dinv = _dinv_from_deg(deg_ref)[:, None]
    w = w_ref[...]
    for t in range(T):
        y_refs[t][...] = dinv * jnp.dot(
            x_ref[t], w, preferred_element_type=jnp.float32
        )


def _make_xw_call():
    return pl.pallas_call(
        _xw_body,
        grid=(N // RB,),
        in_specs=[
            pl.BlockSpec((T, RB, D), lambda i: (0, i, 0)),
            pl.BlockSpec((D, H), lambda i: (0, 0)),
            pl.BlockSpec((NC, RB, DEG_W), lambda i: (0, i, 0)),
        ],
        out_specs=[pl.BlockSpec((RB, H), lambda i: (i, 0)) for _ in range(T)],
        out_shape=[jax.ShapeDtypeStruct((N, H), jnp.float32) for _ in range(T)],
    )


RD = 400


def _lstm_body(s_ref, deg_ref, bg_ref, wih_ref, whh_ref, bih_ref, bhh_ref,
               wfc_ref, bfc_ref, *rest):
    y_refs = rest[:T]
    out_ref = rest[T]
    dinv = _dinv_from_deg(deg_ref)[:, None]
    bg = bg_ref[...]
    wih = wih_ref[...]
    whh = whh_ref[...]
    bias = bih_ref[...] + bhh_ref[...]
    wfc = wfc_ref[...]
    bfc = bfc_ref[0, 0]

    h = jnp.zeros((RD, H), jnp.float32)
    c = jnp.zeros((RD, H), jnp.float32)
    cols = []
    for t in range(T):
        ssum = s_ref[t, 0] + s_ref[t, 1] + y_refs[t][...]
        emb = jnp.maximum(dinv * ssum + bg, 0.0)
        gates = (
            jnp.dot(emb, wih, preferred_element_type=jnp.float32)
            + jnp.dot(h, whh, preferred_element_type=jnp.float32)
            + bias
        )
        gi = jax.nn.sigmoid(gates[:, 0:H])
        gf = jax.nn.sigmoid(gates[:, H:2 * H])
        gg = jnp.tanh(gates[:, 2 * H:3 * H])
        go = jax.nn.sigmoid(gates[:, 3 * H:4 * H])
        c = gf * c + gi * gg
        h = go * jnp.tanh(c)
        cols.append(jnp.dot(h, wfc, preferred_element_type=jnp.float32) + bfc)
    out_ref[...] = jnp.concatenate(cols, axis=1)


def _make_lstm_call():
    return pl.pallas_call(
        _lstm_body,
        grid=(N // RD,),
        in_specs=(
            [
                pl.BlockSpec((T, NC, RD, H), lambda i: (0, 0, i, 0)),

                pl.BlockSpec((NC, RD, DEG_W), lambda i: (0, i, 0)),
                pl.BlockSpec((1, H), lambda i: (0, 0)),
                pl.BlockSpec((H, 4 * H), lambda i: (0, 0)),
                pl.BlockSpec((H, 4 * H), lambda i: (0, 0)),
                pl.BlockSpec((1, 4 * H), lambda i: (0, 0)),
                pl.BlockSpec((1, 4 * H), lambda i: (0, 0)),
                pl.BlockSpec((H, 1), lambda i: (0, 0)),
                pl.BlockSpec((1, 1), lambda i: (0, 0)),
            ]
            + [pl.BlockSpec((RD, H), lambda i: (i, 0)) for _ in range(T)]
        ),
        out_specs=pl.BlockSpec((RD, T), lambda i: (i, 0)),
        out_shape=jax.ShapeDtypeStruct((N, T), jnp.float32),
    )


@jax.jit
def kernel(x_seq, edge_index, W_gcn, b_gcn, W_ih, W_hh, b_ih, b_hh, W_fc, b_fc):
    src = edge_index[0]
    dst = edge_index[1]
    npad = E_PAD - E
    src_p = jnp.concatenate([src, jnp.zeros((npad,), jnp.int32)]).reshape(EROWS, CH)
    dst_p = jnp.concatenate(
        [dst, jnp.full((npad,), PAD_ROW, jnp.int32)]
    ).reshape(EROWS, CH)

    ones16 = jnp.ones((CH, DEG_W), jnp.float32)
    zdeg = jnp.zeros((ZROWS, DEG_W), jnp.float32)
    zrow = jnp.zeros((ZROWS, D), jnp.float32)

    deg = _make_deg_kernel()(dst_p, ones16, zdeg)
    ys = _make_xw_call()(x_seq, W_gcn, deg)
    s_parts = _make_scatter_kernel()(src_p, dst_p, *ys, zrow)
    preds = _make_lstm_call()(
        s_parts,
        deg,
        b_gcn.reshape(1, H),
        W_ih.T,
        W_hh.T,
        b_ih.reshape(1, 4 * H),
        b_hh.reshape(1, 4 * H),
        W_fc.T,
        b_fc.reshape(1, 1),
        *ys,
    )
    return preds.reshape(N, T, 1)
